# baseline (speedup 1.0000x reference)
import functools

import jax
import jax.numpy as jnp
from jax import lax
from jax.experimental import pallas as pl
from jax.experimental.pallas import tpu as pltpu
from jax.experimental.pallas import tpu_sc as plsc

_N = 10000
_HID = 128
_K = 6
_NP1 = 2500
_NP2 = 625
_NP1_PAD = 2560


_NSC = 2
_NSUB = 16
_NW = _NSC * _NSUB


def _sc_segsum_call(table, src_idx, dst_idx, n_out_pad, chunk,
                    nck0=None, nck1=None):
    e_pad = src_idx.shape[0]
    assert e_pad % chunk == 0
    ncks = e_pad // chunk
    if nck0 is None:
        nck0 = nck1 = ncks // _NW
    assert _NSUB * (nck0 + nck1) == ncks
    nck_max = max(nck0, nck1)
    rows_per_sub = n_out_pad // _NSUB
    src2d = src_idx.reshape(ncks, chunk)
    dst2d = dst_idx.reshape(ncks, chunk)
    extra = _NSUB * nck0 + (_NSUB - 1) * nck1 + nck_max - ncks
    if extra > 0:
        src2d = jnp.pad(src2d, ((0, extra), (0, 0)))
        dst2d = jnp.pad(dst2d, ((0, extra), (0, 0)))
    zeros = jnp.zeros((n_out_pad, _HID), jnp.float32)
    mesh = plsc.VectorSubcoreMesh(core_axis_name="c", subcore_axis_name="s")

    nbuf = 2
    use_ring = (nck0 % 2 == 0 and nck1 % 2 == 0 and min(nck0, nck1) >= 4)

    @functools.partial(
        pl.kernel,
        out_type=jax.ShapeDtypeStruct((_NSC, n_out_pad, _HID), jnp.float32),
        mesh=mesh,
        scratch_types=[
            pltpu.VMEM((nck_max, chunk), jnp.int32),
            pltpu.VMEM((nck_max, chunk), jnp.int32),
            pltpu.VMEM((nbuf, chunk, _HID), jnp.float32),
            pltpu.VMEM_SHARED((n_out_pad, _HID), jnp.float32),
            pltpu.SemaphoreType.DMA((nbuf,)),
            pltpu.SemaphoreType.DMA((nbuf,)),
        ],
    )
    def k(table_h, src_h, dst_h, zero_h, out_h, sidx, didx, rows, acc,
          gsem, ssem):
        cid = lax.axis_index("c")
        sid = lax.axis_index("s")
        nck = jnp.where(cid == 0, nck0, nck1)
        base = jnp.where(cid == 0, sid * nck0, _NSUB * nck0 + sid * nck1)
        pltpu.sync_copy(zero_h.at[pl.ds(sid * rows_per_sub, rows_per_sub)],
                        acc.at[pl.ds(sid * rows_per_sub, rows_per_sub)])
        pltpu.sync_copy(src_h.at[pl.ds(base, nck_max)], sidx)
        pltpu.sync_copy(dst_h.at[pl.ds(base, nck_max)], didx)
        plsc.subcore_barrier()

        def gather(c, b):
            pltpu.async_copy(table_h.at[sidx.at[c]], rows.at[b], gsem.at[b])

        def gwait(b):
            pltpu.make_async_copy(table_h.at[sidx.at[0]], rows.at[b],
                                  gsem.at[b]).wait()

        def swait(b):
            pltpu.make_async_copy(rows.at[b], acc.at[didx.at[0]],
                                  ssem.at[b]).wait()

        def scatter(c, b):
            pltpu.async_copy(rows.at[b], acc.at[didx.at[c]], ssem.at[b],
                             add=True)

        if use_ring:
            gather(0, 0)
            gather(1, 1)
            gwait(0)
            scatter(0, 0)

            @pl.loop(1, nck - 1, step=2)
            def _mid(c0):
                for j in range(2):
                    c = c0 + j
                    b = (1 + j) % 2
                    swait(1 - b)
                    gather(c + 1, 1 - b)
                    gwait(b)
                    scatter(c, b)

            gwait(1)
            scatter(nck - 1, 1)
            for b in range(nbuf):
                swait(b)
        else:
            @pl.loop(0, nck)
            def _chunk(c):
                pltpu.async_copy(table_h.at[sidx.at[c]], rows.at[0],
                                 gsem.at[0]).wait()
                pltpu.sync_copy(rows.at[0], acc.at[didx.at[c]], add=True)

        plsc.subcore_barrier()
        pltpu.sync_copy(acc.at[pl.ds(sid * rows_per_sub, rows_per_sub)],
                        out_h.at[cid, pl.ds(sid * rows_per_sub, rows_per_sub)])

    return k(table, src2d, dst2d, zeros)


def _pad_edges(src, dst, e_pad, dump_row, n_dump):
    e = src.shape[0]
    pad = e_pad - e
    src_p = jnp.concatenate([src, jnp.zeros((pad,), jnp.int32)])
    dump = dump_row + (jnp.arange(pad, dtype=jnp.int32) % n_dump)
    dst_p = jnp.concatenate([dst, dump])
    return src_p, dst_p


def _conv_body(x_ref, m_ref, wr_ref, wn_ref, b_ref, o_ref):
    acc = jnp.dot(x_ref[...], wr_ref[...], preferred_element_type=jnp.float32)
    acc = acc + jnp.dot(m_ref[...], wn_ref[...], preferred_element_type=jnp.float32)
    acc = acc + b_ref[...]
    o_ref[...] = jnp.maximum(acc, 0.0)


def _conv2_body(x_ref, ma_ref, mb_ref, wr_ref, wn_ref, b_ref, o_ref):
    acc = jnp.dot(x_ref[...], wr_ref[...], preferred_element_type=jnp.float32)
    m = ma_ref[...] + mb_ref[...]
    acc = acc + jnp.dot(m, wn_ref[...], preferred_element_type=jnp.float32)
    acc = acc + b_ref[...]
    o_ref[...] = jnp.maximum(acc, 0.0)


def _conv_relu(x, msgs, wr, wn, b, block_rows=None):
    n = x.shape[0]
    if block_rows is None:
        block_rows = 512 if n % 512 == 0 else 2000
    assert n % block_rows == 0, n
    grid = n // block_rows
    row_spec = pl.BlockSpec((block_rows, _HID), lambda i: (i, 0))
    w_spec = pl.BlockSpec((_HID, _HID), lambda i: (0, 0))
    b_spec = pl.BlockSpec((1, _HID), lambda i: (0, 0))
    body = _conv_body if len(msgs) == 1 else _conv2_body
    return pl.pallas_call(
        body,
        grid=(grid,),
        in_specs=[row_spec] * (1 + len(msgs)) + [w_spec, w_spec, b_spec],
        out_specs=row_spec,
        out_shape=jax.ShapeDtypeStruct((n, _HID), jnp.float32),
    )(x, *msgs, wr, wn, b.reshape(1, _HID))


def _final_body(x_ref, ma_ref, mb_ref, wr_ref, wn_ref, b_ref, lw_ref, lb_ref,
                o_ref):
    acc = jnp.dot(x_ref[...], wr_ref[...], preferred_element_type=jnp.float32)
    m = ma_ref[...] + mb_ref[...]
    acc = acc + jnp.dot(m, wn_ref[...], preferred_element_type=jnp.float32)
    acc = acc + b_ref[...]
    h = jnp.maximum(acc, 0.0)
    o_ref[...] = jnp.dot(h, lw_ref[...], preferred_element_type=jnp.float32) + lb_ref[...]


def _final_fused(x, msgs, wr, wn, b, lin_w, lin_b, block_rows=None):
    n = x.shape[0]
    if block_rows is None:
        block_rows = 512 if n % 512 == 0 else 2000
    assert n % block_rows == 0, n
    grid = n // block_rows
    row_spec = pl.BlockSpec((block_rows, _HID), lambda i: (i, 0))
    w_spec = pl.BlockSpec((_HID, _HID), lambda i: (0, 0))
    b_spec = pl.BlockSpec((1, _HID), lambda i: (0, 0))
    return pl.pallas_call(
        _final_body,
        grid=(grid,),
        in_specs=[row_spec, row_spec, row_spec, w_spec, w_spec, b_spec,
                  w_spec, b_spec],
        out_specs=row_spec,
        out_shape=jax.ShapeDtypeStruct((n, _HID), jnp.float32),
    )(x, *msgs, wr, wn, b.reshape(1, _HID), lin_w, lin_b.reshape(1, _HID))


def _knn_body(pos_ref, post_ref, o_ref, *, block_rows, npts_pad):
    i = pl.program_id(0)
    pr = pos_ref[...]
    pt = post_ref[...]
    d = jnp.zeros((block_rows, npts_pad), jnp.float32)
    for c in range(3):
        diff = pr[:, c:c + 1] - pt[c:c + 1, :]
        d = d + diff * diff
    row_id = lax.broadcasted_iota(jnp.int32, (block_rows, npts_pad), 0) + i * block_rows
    col_id = lax.broadcasted_iota(jnp.int32, (block_rows, npts_pad), 1)
    d = jnp.where(row_id == col_id, d + 1e10, d)
    cols = []
    for _ in range(_K):
        a = jnp.argmin(d, axis=1).astype(jnp.int32)
        cols.append(a[:, None])
        d = jnp.where(col_id == a[:, None], jnp.inf, d)
    cols.append(jnp.zeros((block_rows, 2), jnp.int32))
    o_ref[...] = jnp.concatenate(cols, axis=1)


def _knn_top6(pos_pad, post_pad, block_rows=256):
    npts_pad = post_pad.shape[1]
    grid = npts_pad // block_rows
    return pl.pallas_call(
        functools.partial(_knn_body, block_rows=block_rows, npts_pad=npts_pad),
        grid=(grid,),
        in_specs=[
            pl.BlockSpec((block_rows, 8), lambda i: (i, 0)),
            pl.BlockSpec((8, npts_pad), lambda i: (0, 0)),
        ],
        out_specs=pl.BlockSpec((block_rows, 8), lambda i: (i, 0)),
        out_shape=jax.ShapeDtypeStruct((npts_pad, 8), jnp.int32),
    )(pos_pad, post_pad)


def kernel(x, pos, edge_index, batch, d0_wr, d0_wn, d0_b, d1_wr, d1_wn, d1_b,
           u0_wr, u0_wn, u0_b, u1_wr, u1_wn, u1_b, lin_w, lin_b):
    n = x.shape[0]
    src0, dst0 = edge_index[0], edge_index[1]

    def _constants():
        p0 = jax.random.permutation(jax.random.key(100), n)[:_NP1]
        p1 = jax.random.permutation(jax.random.key(101), _NP1)[:_NP2]
        i1 = jnp.full((_NP1,), _NP2, jnp.int32).at[p1].set(
            jnp.arange(_NP2, dtype=jnp.int32))
        i0 = jnp.full((n,), _NP1, jnp.int32).at[p0].set(
            jnp.arange(_NP1, dtype=jnp.int32))
        dk = jnp.repeat(jnp.arange(_NP1, dtype=jnp.int32), _K)
        return p0.astype(jnp.int32), p1.astype(jnp.int32), i0, i1, dk
    try:
        with jax.ensure_compile_time_eval(), \
                jax.default_device(jax.devices("cpu")[0]):
            perm0, perm1, inv0, inv1, dstk = _constants()
    except Exception:
        perm0, perm1, inv0, inv1, dstk = _constants()

    n_acc = 10112
    e_pad0 = 163840
    src0_p, dst0_p = _pad_edges(src0, dst0, e_pad0, n, n_acc - n)
    parts0 = _sc_segsum_call(x, src0_p, dst0_p, n_acc, 128, 16, 64)
    h0 = _conv_relu(x, [parts0[0, :n], parts0[1, :n]], d0_wr, d0_wn, d0_b)

    pos1 = pos[perm0]
    pos1_pad = jnp.full((_NP1_PAD, 8), 1e6, jnp.float32)
    pos1_pad = pos1_pad.at[:_NP1, :3].set(pos1)
    post_pad = pos1_pad.T.reshape(8, _NP1_PAD) + 0.0
    idx_pad = _knn_top6(pos1_pad, post_pad)
    idx1 = idx_pad[:_NP1, :_K]

    e_pad1 = 16384
    gidx1 = perm0[idx1.reshape(-1)]
    srck, dstk_p = _pad_edges(gidx1, dstk, e_pad1, _NP1, _NP1_PAD - _NP1)
    parts1 = _sc_segsum_call(h0, srck, dstk_p, _NP1_PAD, 64)
    x1 = h0[perm0]
    h1 = _conv_relu(
        jnp.pad(x1, ((0, _NP1_PAD - _NP1), (0, 0))),
        [parts1[0], parts1[1]],
        d1_wr, d1_wn, d1_b)[:_NP1]

    x2 = h1[perm1]
    t_tab = jnp.concatenate([x2, jnp.zeros((1, _HID), jnp.float32)], axis=0)
    xr_up0 = t_tab[inv1]
    srcu = inv1[idx1.reshape(-1)]
    srcu_p, _ = _pad_edges(srcu, dstk, e_pad1, _NP1, _NP1_PAD - _NP1)
    parts2 = _sc_segsum_call(t_tab, srcu_p, dstk_p, _NP1_PAD, 64)
    h2 = _conv_relu(
        jnp.pad(xr_up0, ((0, _NP1_PAD - _NP1), (0, 0))),
        [parts2[0], parts2[1]],
        u0_wr, u0_wn, u0_b)[:_NP1]

    u_tab = jnp.concatenate([h2, jnp.zeros((1, _HID), jnp.float32)], axis=0)
    xg = u_tab[inv0]
    parts3 = _sc_segsum_call(xg, src0_p, dst0_p, n_acc, 128, 16, 64)
    return _final_fused(xg, [parts3[0, :n], parts3[1, :n]],
                        u1_wr, u1_wn, u1_b, lin_w, lin_b)


# speedup vs baseline: 1.0376x; 1.0376x over previous
import functools

import jax
import jax.numpy as jnp
from jax import lax
from jax.experimental import pallas as pl
from jax.experimental.pallas import tpu as pltpu
from jax.experimental.pallas import tpu_sc as plsc

_N = 10000
_HID = 128
_K = 6
_NP1 = 2500
_NP2 = 625
_NP1_PAD = 2560


_NSC = 2
_NSUB = 16
_NW = _NSC * _NSUB


def _sc_segsum_call(table, src_idx, dst_idx, n_out_pad, chunk,
                    nck0=None, nck1=None):
    e_pad = src_idx.shape[0]
    assert e_pad % chunk == 0
    ncks = e_pad // chunk
    if nck0 is None:
        nck0 = nck1 = ncks // _NW
    assert _NSUB * (nck0 + nck1) == ncks
    nck_max = max(nck0, nck1)
    rows_per_sub = n_out_pad // _NSUB
    src2d = src_idx.reshape(ncks, chunk)
    dst2d = dst_idx.reshape(ncks, chunk)
    extra = _NSUB * nck0 + (_NSUB - 1) * nck1 + nck_max - ncks
    if extra > 0:
        src2d = jnp.pad(src2d, ((0, extra), (0, 0)))
        dst2d = jnp.pad(dst2d, ((0, extra), (0, 0)))
    zeros = jnp.zeros((n_out_pad, _HID), jnp.float32)
    mesh = plsc.VectorSubcoreMesh(core_axis_name="c", subcore_axis_name="s")

    nbuf = 2
    use_ring = (nck0 % 2 == 0 and nck1 % 2 == 0 and min(nck0, nck1) >= 4)

    @functools.partial(
        pl.kernel,
        out_type=jax.ShapeDtypeStruct((_NSC, n_out_pad, _HID), jnp.float32),
        mesh=mesh,
        scratch_types=[
            pltpu.VMEM((nck_max, chunk), jnp.int32),
            pltpu.VMEM((nck_max, chunk), jnp.int32),
            pltpu.VMEM((nbuf, chunk, _HID), jnp.float32),
            pltpu.VMEM_SHARED((n_out_pad, _HID), jnp.float32),
            pltpu.SemaphoreType.DMA((nbuf,)),
            pltpu.SemaphoreType.DMA((nbuf,)),
        ],
    )
    def k(table_h, src_h, dst_h, zero_h, out_h, sidx, didx, rows, acc,
          gsem, ssem):
        cid = lax.axis_index("c")
        sid = lax.axis_index("s")
        nck = jnp.where(cid == 0, nck0, nck1)
        base = jnp.where(cid == 0, sid * nck0, _NSUB * nck0 + sid * nck1)
        pltpu.sync_copy(zero_h.at[pl.ds(sid * rows_per_sub, rows_per_sub)],
                        acc.at[pl.ds(sid * rows_per_sub, rows_per_sub)])
        pltpu.sync_copy(src_h.at[pl.ds(base, nck_max)], sidx)
        pltpu.sync_copy(dst_h.at[pl.ds(base, nck_max)], didx)
        plsc.subcore_barrier()

        def gather(c, b):
            pltpu.async_copy(table_h.at[sidx.at[c]], rows.at[b], gsem.at[b])

        def gwait(b):
            pltpu.make_async_copy(table_h.at[sidx.at[0]], rows.at[b],
                                  gsem.at[b]).wait()

        def swait(b):
            pltpu.make_async_copy(rows.at[b], acc.at[didx.at[0]],
                                  ssem.at[b]).wait()

        def scatter(c, b):
            pltpu.async_copy(rows.at[b], acc.at[didx.at[c]], ssem.at[b],
                             add=True)

        if use_ring:
            gather(0, 0)
            gather(1, 1)
            gwait(0)
            scatter(0, 0)

            @pl.loop(1, nck - 1, step=2)
            def _mid(c0):
                for j in range(2):
                    c = c0 + j
                    b = (1 + j) % 2
                    swait(1 - b)
                    gather(c + 1, 1 - b)
                    gwait(b)
                    scatter(c, b)

            gwait(1)
            scatter(nck - 1, 1)
            for b in range(nbuf):
                swait(b)
        else:
            @pl.loop(0, nck)
            def _chunk(c):
                pltpu.async_copy(table_h.at[sidx.at[c]], rows.at[0],
                                 gsem.at[0]).wait()
                pltpu.sync_copy(rows.at[0], acc.at[didx.at[c]], add=True)

        plsc.subcore_barrier()
        pltpu.sync_copy(acc.at[pl.ds(sid * rows_per_sub, rows_per_sub)],
                        out_h.at[cid, pl.ds(sid * rows_per_sub, rows_per_sub)])

    return k(table, src2d, dst2d, zeros)


def _pad_edges(src, dst, e_pad, dump_row, n_dump):
    e = src.shape[0]
    pad = e_pad - e
    src_p = jnp.concatenate([src, jnp.zeros((pad,), jnp.int32)])
    dump = dump_row + (jnp.arange(pad, dtype=jnp.int32) % n_dump)
    dst_p = jnp.concatenate([dst, dump])
    return src_p, dst_p


def _conv_body(x_ref, m_ref, wr_ref, wn_ref, b_ref, o_ref):
    acc = jnp.dot(x_ref[...], wr_ref[...], preferred_element_type=jnp.float32)
    acc = acc + jnp.dot(m_ref[...], wn_ref[...], preferred_element_type=jnp.float32)
    acc = acc + b_ref[...]
    o_ref[...] = jnp.maximum(acc, 0.0)


def _conv2_body(x_ref, ma_ref, mb_ref, wr_ref, wn_ref, b_ref, o_ref):
    acc = jnp.dot(x_ref[...], wr_ref[...], preferred_element_type=jnp.float32)
    m = ma_ref[...] + mb_ref[...]
    acc = acc + jnp.dot(m, wn_ref[...], preferred_element_type=jnp.float32)
    acc = acc + b_ref[...]
    o_ref[...] = jnp.maximum(acc, 0.0)


def _conv_relu(x, msgs, wr, wn, b, block_rows=None):
    n = x.shape[0]
    if block_rows is None:
        block_rows = 512 if n % 512 == 0 else 2000
    assert n % block_rows == 0, n
    grid = n // block_rows
    row_spec = pl.BlockSpec((block_rows, _HID), lambda i: (i, 0))
    w_spec = pl.BlockSpec((_HID, _HID), lambda i: (0, 0))
    b_spec = pl.BlockSpec((1, _HID), lambda i: (0, 0))
    body = _conv_body if len(msgs) == 1 else _conv2_body
    return pl.pallas_call(
        body,
        grid=(grid,),
        in_specs=[row_spec] * (1 + len(msgs)) + [w_spec, w_spec, b_spec],
        out_specs=row_spec,
        out_shape=jax.ShapeDtypeStruct((n, _HID), jnp.float32),
    )(x, *msgs, wr, wn, b.reshape(1, _HID))


def _final_body(x_ref, ma_ref, mb_ref, wr_ref, wn_ref, b_ref, lw_ref, lb_ref,
                o_ref):
    acc = jnp.dot(x_ref[...], wr_ref[...], preferred_element_type=jnp.float32)
    m = ma_ref[...] + mb_ref[...]
    acc = acc + jnp.dot(m, wn_ref[...], preferred_element_type=jnp.float32)
    acc = acc + b_ref[...]
    h = jnp.maximum(acc, 0.0)
    o_ref[...] = jnp.dot(h, lw_ref[...], preferred_element_type=jnp.float32) + lb_ref[...]


def _final_fused(x, msgs, wr, wn, b, lin_w, lin_b, block_rows=None):
    n = x.shape[0]
    if block_rows is None:
        block_rows = 512 if n % 512 == 0 else 2000
    assert n % block_rows == 0, n
    grid = n // block_rows
    row_spec = pl.BlockSpec((block_rows, _HID), lambda i: (i, 0))
    w_spec = pl.BlockSpec((_HID, _HID), lambda i: (0, 0))
    b_spec = pl.BlockSpec((1, _HID), lambda i: (0, 0))
    return pl.pallas_call(
        _final_body,
        grid=(grid,),
        in_specs=[row_spec, row_spec, row_spec, w_spec, w_spec, b_spec,
                  w_spec, b_spec],
        out_specs=row_spec,
        out_shape=jax.ShapeDtypeStruct((n, _HID), jnp.float32),
    )(x, *msgs, wr, wn, b.reshape(1, _HID), lin_w, lin_b.reshape(1, _HID))


def _knn_body(pos_ref, post_ref, o_ref, *, block_rows, npts_pad):
    i = pl.program_id(0)
    pr = pos_ref[...]
    pt = post_ref[...]
    d = jnp.zeros((block_rows, npts_pad), jnp.float32)
    for c in range(3):
        diff = pr[:, c:c + 1] - pt[c:c + 1, :]
        d = d + diff * diff
    row_id = lax.broadcasted_iota(jnp.int32, (block_rows, npts_pad), 0) + i * block_rows
    col_id = lax.broadcasted_iota(jnp.int32, (block_rows, npts_pad), 1)
    d = jnp.where(row_id == col_id, d + 1e10, d)
    cols = []
    for _ in range(_K):
        a = jnp.argmin(d, axis=1).astype(jnp.int32)
        cols.append(a[:, None])
        d = jnp.where(col_id == a[:, None], jnp.inf, d)
    cols.append(jnp.zeros((block_rows, 2), jnp.int32))
    o_ref[...] = jnp.concatenate(cols, axis=1)


def _knn_top6(pos_pad, post_pad, block_rows=256):
    npts_pad = post_pad.shape[1]
    grid = npts_pad // block_rows
    return pl.pallas_call(
        functools.partial(_knn_body, block_rows=block_rows, npts_pad=npts_pad),
        grid=(grid,),
        in_specs=[
            pl.BlockSpec((block_rows, 8), lambda i: (i, 0)),
            pl.BlockSpec((8, npts_pad), lambda i: (0, 0)),
        ],
        out_specs=pl.BlockSpec((block_rows, 8), lambda i: (i, 0)),
        out_shape=jax.ShapeDtypeStruct((npts_pad, 8), jnp.int32),
    )(pos_pad, post_pad)


def kernel(x, pos, edge_index, batch, d0_wr, d0_wn, d0_b, d1_wr, d1_wn, d1_b,
           u0_wr, u0_wn, u0_b, u1_wr, u1_wn, u1_b, lin_w, lin_b):
    n = x.shape[0]
    src0, dst0 = edge_index[0], edge_index[1]

    def _constants():
        p0 = jax.random.permutation(jax.random.key(100), n)[:_NP1]
        p1 = jax.random.permutation(jax.random.key(101), _NP1)[:_NP2]
        i1 = jnp.full((_NP1,), _NP2, jnp.int32).at[p1].set(
            jnp.arange(_NP2, dtype=jnp.int32))
        i0 = jnp.full((n,), _NP1, jnp.int32).at[p0].set(
            jnp.arange(_NP1, dtype=jnp.int32))
        dk = jnp.repeat(jnp.arange(_NP1, dtype=jnp.int32), _K)
        return p0.astype(jnp.int32), p1.astype(jnp.int32), i0, i1, dk
    try:
        with jax.ensure_compile_time_eval(), \
                jax.default_device(jax.devices("cpu")[0]):
            perm0, perm1, inv0, inv1, dstk = _constants()
    except Exception:
        perm0, perm1, inv0, inv1, dstk = _constants()

    n_acc = 10112
    e_pad0 = 163840
    src0_p, dst0_p = _pad_edges(src0, dst0, e_pad0, n, n_acc - n)
    parts0 = _sc_segsum_call(x, src0_p, dst0_p, n_acc, 64)
    h0 = _conv_relu(x, [parts0[0, :n], parts0[1, :n]], d0_wr, d0_wn, d0_b)

    pos1 = pos[perm0]
    pos1_pad = jnp.full((_NP1_PAD, 8), 1e6, jnp.float32)
    pos1_pad = pos1_pad.at[:_NP1, :3].set(pos1)
    post_pad = pos1_pad.T.reshape(8, _NP1_PAD) + 0.0
    idx_pad = _knn_top6(pos1_pad, post_pad)
    idx1 = idx_pad[:_NP1, :_K]

    e_pad1 = 16384
    gidx1 = perm0[idx1.reshape(-1)]
    srck, dstk_p = _pad_edges(gidx1, dstk, e_pad1, _NP1, _NP1_PAD - _NP1)
    parts1 = _sc_segsum_call(h0, srck, dstk_p, _NP1_PAD, 64)
    x1 = h0[perm0]
    h1 = _conv_relu(
        jnp.pad(x1, ((0, _NP1_PAD - _NP1), (0, 0))),
        [parts1[0], parts1[1]],
        d1_wr, d1_wn, d1_b)[:_NP1]

    x2 = h1[perm1]
    t_tab = jnp.concatenate([x2, jnp.zeros((1, _HID), jnp.float32)], axis=0)
    xr_up0 = t_tab[inv1]
    srcu = inv1[idx1.reshape(-1)]
    srcu_p, _ = _pad_edges(srcu, dstk, e_pad1, _NP1, _NP1_PAD - _NP1)
    parts2 = _sc_segsum_call(t_tab, srcu_p, dstk_p, _NP1_PAD, 64)
    h2 = _conv_relu(
        jnp.pad(xr_up0, ((0, _NP1_PAD - _NP1), (0, 0))),
        [parts2[0], parts2[1]],
        u0_wr, u0_wn, u0_b)[:_NP1]

    u_tab = jnp.concatenate([h2, jnp.zeros((1, _HID), jnp.float32)], axis=0)
    xg = u_tab[inv0]
    parts3 = _sc_segsum_call(xg, src0_p, dst0_p, n_acc, 64)
    return _final_fused(xg, [parts3[0, :n], parts3[1, :n]],
                        u1_wr, u1_wn, u1_b, lin_w, lin_b)


# speedup vs baseline: 1.0400x; 1.0024x over previous
import functools

import jax
import jax.numpy as jnp
from jax import lax
from jax.experimental import pallas as pl
from jax.experimental.pallas import tpu as pltpu
from jax.experimental.pallas import tpu_sc as plsc

_N = 10000
_HID = 128
_K = 6
_NP1 = 2500
_NP2 = 625
_NP1_PAD = 2560


_NSC = 2
_NSUB = 16
_NW = _NSC * _NSUB


def _sc_segsum_call(table, src_idx, dst_idx, n_out_pad, chunk,
                    nck0=None, nck1=None, gidx=None):
    e_pad = src_idx.shape[0]
    assert e_pad % chunk == 0
    ncks = e_pad // chunk
    if nck0 is None:
        nck0 = nck1 = ncks // _NW
    assert _NSUB * (nck0 + nck1) == ncks
    nck_max = max(nck0, nck1)
    rows_per_sub = n_out_pad // _NSUB
    src2d = src_idx.reshape(ncks, chunk)
    dst2d = dst_idx.reshape(ncks, chunk)
    extra = _NSUB * nck0 + (_NSUB - 1) * nck1 + nck_max - ncks
    if extra > 0:
        src2d = jnp.pad(src2d, ((0, extra), (0, 0)))
        dst2d = jnp.pad(dst2d, ((0, extra), (0, 0)))
    zeros = jnp.zeros((n_out_pad, _HID), jnp.float32)
    mesh = plsc.VectorSubcoreMesh(core_axis_name="c", subcore_axis_name="s")

    nbuf = 2
    use_ring = (nck0 % 2 == 0 and nck1 % 2 == 0 and min(nck0, nck1) >= 4)

    out_types = [jax.ShapeDtypeStruct((_NSC, n_out_pad, _HID), jnp.float32)]
    scratch = [
        pltpu.VMEM((nck_max, chunk), jnp.int32),
        pltpu.VMEM((nck_max, chunk), jnp.int32),
        pltpu.VMEM((nbuf, chunk, _HID), jnp.float32),
        pltpu.VMEM_SHARED((n_out_pad, _HID), jnp.float32),
        pltpu.SemaphoreType.DMA((nbuf,)),
        pltpu.SemaphoreType.DMA((nbuf,)),
    ]
    grows = n_out_pad // _NW
    if gidx is not None:
        out_types.append(jax.ShapeDtypeStruct((n_out_pad, _HID), jnp.float32))
        scratch += [
            pltpu.VMEM((1, grows), jnp.int32),
            pltpu.VMEM((grows, _HID), jnp.float32),
            pltpu.SemaphoreType.DMA,
        ]
        gidx2d = gidx.reshape(_NW, grows)

    @functools.partial(
        pl.kernel,
        out_type=tuple(out_types) if gidx is not None else out_types[0],
        mesh=mesh,
        scratch_types=scratch,
    )
    def k(table_h, src_h, dst_h, zero_h, *rest):
        if gidx is not None:
            (gidx_h, out_h, gout_h, sidx, didx, rows, acc, gsem, ssem,
             gvidx, grbuf, g2sem) = rest
        else:
            out_h, sidx, didx, rows, acc, gsem, ssem = rest
        cid = lax.axis_index("c")
        sid = lax.axis_index("s")
        wid = cid * _NSUB + sid
        nck = jnp.where(cid == 0, nck0, nck1)
        base = jnp.where(cid == 0, sid * nck0, _NSUB * nck0 + sid * nck1)
        pltpu.sync_copy(zero_h.at[pl.ds(sid * rows_per_sub, rows_per_sub)],
                        acc.at[pl.ds(sid * rows_per_sub, rows_per_sub)])
        pltpu.sync_copy(src_h.at[pl.ds(base, nck_max)], sidx)
        pltpu.sync_copy(dst_h.at[pl.ds(base, nck_max)], didx)
        if gidx is not None:
            pltpu.sync_copy(gidx_h.at[pl.ds(wid, 1)], gvidx)
            pltpu.async_copy(table_h.at[gvidx.at[0]], grbuf, g2sem).wait()
            pltpu.sync_copy(grbuf, gout_h.at[pl.ds(wid * grows, grows)])
        plsc.subcore_barrier()

        def gather(c, b):
            pltpu.async_copy(table_h.at[sidx.at[c]], rows.at[b], gsem.at[b])

        def gwait(b):
            pltpu.make_async_copy(table_h.at[sidx.at[0]], rows.at[b],
                                  gsem.at[b]).wait()

        def swait(b):
            pltpu.make_async_copy(rows.at[b], acc.at[didx.at[0]],
                                  ssem.at[b]).wait()

        def scatter(c, b):
            pltpu.async_copy(rows.at[b], acc.at[didx.at[c]], ssem.at[b],
                             add=True)

        if use_ring:
            gather(0, 0)
            gather(1, 1)
            gwait(0)
            scatter(0, 0)

            @pl.loop(1, nck - 1, step=2)
            def _mid(c0):
                for j in range(2):
                    c = c0 + j
                    b = (1 + j) % 2
                    swait(1 - b)
                    gather(c + 1, 1 - b)
                    gwait(b)
                    scatter(c, b)

            gwait(1)
            scatter(nck - 1, 1)
            for b in range(nbuf):
                swait(b)
        else:
            @pl.loop(0, nck)
            def _chunk(c):
                pltpu.async_copy(table_h.at[sidx.at[c]], rows.at[0],
                                 gsem.at[0]).wait()
                pltpu.sync_copy(rows.at[0], acc.at[didx.at[c]], add=True)

        plsc.subcore_barrier()
        pltpu.sync_copy(acc.at[pl.ds(sid * rows_per_sub, rows_per_sub)],
                        out_h.at[cid, pl.ds(sid * rows_per_sub, rows_per_sub)])

    if gidx is not None:
        return k(table, src2d, dst2d, zeros, gidx2d)
    return k(table, src2d, dst2d, zeros)


def _pad_edges(src, dst, e_pad, dump_row, n_dump):
    e = src.shape[0]
    pad = e_pad - e
    src_p = jnp.concatenate([src, jnp.zeros((pad,), jnp.int32)])
    dump = dump_row + (jnp.arange(pad, dtype=jnp.int32) % n_dump)
    dst_p = jnp.concatenate([dst, dump])
    return src_p, dst_p


def _conv_body(x_ref, m_ref, wr_ref, wn_ref, b_ref, o_ref):
    acc = jnp.dot(x_ref[...], wr_ref[...], preferred_element_type=jnp.float32)
    acc = acc + jnp.dot(m_ref[...], wn_ref[...], preferred_element_type=jnp.float32)
    acc = acc + b_ref[...]
    o_ref[...] = jnp.maximum(acc, 0.0)


def _conv2_body(x_ref, ma_ref, mb_ref, wr_ref, wn_ref, b_ref, o_ref):
    acc = jnp.dot(x_ref[...], wr_ref[...], preferred_element_type=jnp.float32)
    m = ma_ref[...] + mb_ref[...]
    acc = acc + jnp.dot(m, wn_ref[...], preferred_element_type=jnp.float32)
    acc = acc + b_ref[...]
    o_ref[...] = jnp.maximum(acc, 0.0)


def _conv_relu(x, msgs, wr, wn, b, block_rows=None):
    n = x.shape[0]
    if block_rows is None:
        block_rows = 512 if n % 512 == 0 else 2000
    assert n % block_rows == 0, n
    grid = n // block_rows
    row_spec = pl.BlockSpec((block_rows, _HID), lambda i: (i, 0))
    w_spec = pl.BlockSpec((_HID, _HID), lambda i: (0, 0))
    b_spec = pl.BlockSpec((1, _HID), lambda i: (0, 0))
    body = _conv_body if len(msgs) == 1 else _conv2_body
    return pl.pallas_call(
        body,
        grid=(grid,),
        in_specs=[row_spec] * (1 + len(msgs)) + [w_spec, w_spec, b_spec],
        out_specs=row_spec,
        out_shape=jax.ShapeDtypeStruct((n, _HID), jnp.float32),
    )(x, *msgs, wr, wn, b.reshape(1, _HID))


def _final_body(x_ref, ma_ref, mb_ref, wr_ref, wn_ref, b_ref, lw_ref, lb_ref,
                o_ref):
    acc = jnp.dot(x_ref[...], wr_ref[...], preferred_element_type=jnp.float32)
    m = ma_ref[...] + mb_ref[...]
    acc = acc + jnp.dot(m, wn_ref[...], preferred_element_type=jnp.float32)
    acc = acc + b_ref[...]
    h = jnp.maximum(acc, 0.0)
    o_ref[...] = jnp.dot(h, lw_ref[...], preferred_element_type=jnp.float32) + lb_ref[...]


def _final_fused(x, msgs, wr, wn, b, lin_w, lin_b, block_rows=None):
    n = x.shape[0]
    if block_rows is None:
        block_rows = 512 if n % 512 == 0 else 2000
    assert n % block_rows == 0, n
    grid = n // block_rows
    row_spec = pl.BlockSpec((block_rows, _HID), lambda i: (i, 0))
    w_spec = pl.BlockSpec((_HID, _HID), lambda i: (0, 0))
    b_spec = pl.BlockSpec((1, _HID), lambda i: (0, 0))
    return pl.pallas_call(
        _final_body,
        grid=(grid,),
        in_specs=[row_spec, row_spec, row_spec, w_spec, w_spec, b_spec,
                  w_spec, b_spec],
        out_specs=row_spec,
        out_shape=jax.ShapeDtypeStruct((n, _HID), jnp.float32),
    )(x, *msgs, wr, wn, b.reshape(1, _HID), lin_w, lin_b.reshape(1, _HID))


def _knn_body(pos_ref, post_ref, o_ref, *, block_rows, npts_pad):
    i = pl.program_id(0)
    pr = pos_ref[...]
    pt = post_ref[...]
    d = jnp.zeros((block_rows, npts_pad), jnp.float32)
    for c in range(3):
        diff = pr[:, c:c + 1] - pt[c:c + 1, :]
        d = d + diff * diff
    row_id = lax.broadcasted_iota(jnp.int32, (block_rows, npts_pad), 0) + i * block_rows
    col_id = lax.broadcasted_iota(jnp.int32, (block_rows, npts_pad), 1)
    d = jnp.where(row_id == col_id, d + 1e10, d)
    cols = []
    for _ in range(_K):
        a = jnp.argmin(d, axis=1).astype(jnp.int32)
        cols.append(a[:, None])
        d = jnp.where(col_id == a[:, None], jnp.inf, d)
    cols.append(jnp.zeros((block_rows, 2), jnp.int32))
    o_ref[...] = jnp.concatenate(cols, axis=1)


def _knn_top6(pos_pad, post_pad, block_rows=256):
    npts_pad = post_pad.shape[1]
    grid = npts_pad // block_rows
    return pl.pallas_call(
        functools.partial(_knn_body, block_rows=block_rows, npts_pad=npts_pad),
        grid=(grid,),
        in_specs=[
            pl.BlockSpec((block_rows, 8), lambda i: (i, 0)),
            pl.BlockSpec((8, npts_pad), lambda i: (0, 0)),
        ],
        out_specs=pl.BlockSpec((block_rows, 8), lambda i: (i, 0)),
        out_shape=jax.ShapeDtypeStruct((npts_pad, 8), jnp.int32),
    )(pos_pad, post_pad)


def kernel(x, pos, edge_index, batch, d0_wr, d0_wn, d0_b, d1_wr, d1_wn, d1_b,
           u0_wr, u0_wn, u0_b, u1_wr, u1_wn, u1_b, lin_w, lin_b):
    n = x.shape[0]
    src0, dst0 = edge_index[0], edge_index[1]

    def _constants():
        p0 = jax.random.permutation(jax.random.key(100), n)[:_NP1]
        p1 = jax.random.permutation(jax.random.key(101), _NP1)[:_NP2]
        i1 = jnp.full((_NP1,), _NP2, jnp.int32).at[p1].set(
            jnp.arange(_NP2, dtype=jnp.int32))
        i0 = jnp.full((n,), _NP1, jnp.int32).at[p0].set(
            jnp.arange(_NP1, dtype=jnp.int32))
        dk = jnp.tile(jnp.arange(_NP1, dtype=jnp.int32), _K)
        perm0_pad = jnp.concatenate(
            [p0.astype(jnp.int32), jnp.zeros((_NP1_PAD - _NP1,), jnp.int32)])
        inv1_pad = jnp.concatenate(
            [i1, jnp.full((_NP1_PAD - _NP1,), _NP2, jnp.int32)])
        return (p0.astype(jnp.int32), p1.astype(jnp.int32), i0, i1, dk,
                perm0_pad, inv1_pad)
    try:
        with jax.ensure_compile_time_eval(), \
                jax.default_device(jax.devices("cpu")[0]):
            perm0, perm1, inv0, inv1, dstk, perm0_pad, inv1_pad = _constants()
    except Exception:
        perm0, perm1, inv0, inv1, dstk, perm0_pad, inv1_pad = _constants()

    n_acc = 10112
    e_pad0 = 163840
    src0_p, dst0_p = _pad_edges(src0, dst0, e_pad0, n, n_acc - n)
    parts0 = _sc_segsum_call(x, src0_p, dst0_p, n_acc, 64)
    h0 = _conv_relu(x, [parts0[0, :n], parts0[1, :n]], d0_wr, d0_wn, d0_b)

    pos1 = pos[perm0]
    pos1_pad = jnp.full((_NP1_PAD, 8), 1e6, jnp.float32)
    pos1_pad = pos1_pad.at[:_NP1, :3].set(pos1)
    post_pad = pos1_pad.T.reshape(8, _NP1_PAD) + 0.0
    idx_pad = _knn_top6(pos1_pad, post_pad)
    idx1 = idx_pad[:_NP1, :_K]

    e_pad1 = 16384
    idx1t = idx1.T.reshape(-1)
    gidx1 = perm0[idx1t]
    srck, dstk_p = _pad_edges(gidx1, dstk, e_pad1, _NP1, _NP1_PAD - _NP1)
    parts1, x1p = _sc_segsum_call(h0, srck, dstk_p, _NP1_PAD, 64,
                                  gidx=perm0_pad)
    h1 = _conv_relu(x1p, [parts1[0], parts1[1]], d1_wr, d1_wn, d1_b)[:_NP1]

    x2 = h1[perm1]
    t_tab = jnp.concatenate([x2, jnp.zeros((1, _HID), jnp.float32)], axis=0)
    srcu = inv1[idx1t]
    srcu_p, _ = _pad_edges(srcu, dstk, e_pad1, _NP1, _NP1_PAD - _NP1)
    parts2, xr0p = _sc_segsum_call(t_tab, srcu_p, dstk_p, _NP1_PAD, 64,
                                   gidx=inv1_pad)
    h2 = _conv_relu(xr0p, [parts2[0], parts2[1]], u0_wr, u0_wn, u0_b)[:_NP1]

    u_tab = jnp.concatenate([h2, jnp.zeros((1, _HID), jnp.float32)], axis=0)
    xg = u_tab[inv0]
    parts3 = _sc_segsum_call(xg, src0_p, dst0_p, n_acc, 64)
    return _final_fused(xg, [parts3[0, :n], parts3[1, :n]],
                        u1_wr, u1_wn, u1_b, lin_w, lin_b)


# speedup vs baseline: 1.7678x; 1.6998x over previous
import functools

import jax
import jax.numpy as jnp
from jax import lax
from jax.experimental import pallas as pl
from jax.experimental.pallas import tpu as pltpu
from jax.experimental.pallas import tpu_sc as plsc

_N = 10000
_HID = 128
_K = 6
_NP1 = 2500
_NP2 = 625
_NP1_PAD = 2560


_NSC = 2
_NSUB = 16
_NW = _NSC * _NSUB


def _sc_segsum_call(table, src_idx, dst_idx, n_out_pad, chunk,
                    nck0=None, nck1=None, gidx=None):
    e_pad = src_idx.shape[0]
    assert e_pad % chunk == 0
    ncks = e_pad // chunk
    if nck0 is None:
        nck0 = nck1 = ncks // _NW
    assert _NSUB * (nck0 + nck1) == ncks
    nck_max = max(nck0, nck1)
    rows_per_sub = n_out_pad // _NSUB
    src2d = src_idx.reshape(ncks, chunk)
    dst2d = dst_idx.reshape(ncks, chunk)
    extra = _NSUB * nck0 + (_NSUB - 1) * nck1 + nck_max - ncks
    if extra > 0:
        src2d = jnp.pad(src2d, ((0, extra), (0, 0)))
        dst2d = jnp.pad(dst2d, ((0, extra), (0, 0)))
    zeros = jnp.zeros((n_out_pad, _HID), jnp.float32)
    mesh = plsc.VectorSubcoreMesh(core_axis_name="c", subcore_axis_name="s")

    nbuf = 2
    use_ring = (nck0 % 2 == 0 and nck1 % 2 == 0 and min(nck0, nck1) >= 4)

    out_types = [jax.ShapeDtypeStruct((_NSC, n_out_pad, _HID), jnp.float32)]
    scratch = [
        pltpu.VMEM((nck_max, chunk), jnp.int32),
        pltpu.VMEM((nck_max, chunk), jnp.int32),
        pltpu.VMEM((nbuf, chunk, _HID), jnp.float32),
        pltpu.VMEM_SHARED((n_out_pad, _HID), jnp.float32),
        pltpu.SemaphoreType.DMA((nbuf,)),
        pltpu.SemaphoreType.DMA((nbuf,)),
    ]
    grows = n_out_pad // _NW
    if gidx is not None:
        out_types.append(jax.ShapeDtypeStruct((n_out_pad, _HID), jnp.float32))
        scratch += [
            pltpu.VMEM((1, grows), jnp.int32),
            pltpu.VMEM((grows, _HID), jnp.float32),
            pltpu.SemaphoreType.DMA,
        ]
        gidx2d = gidx.reshape(_NW, grows)

    @functools.partial(
        pl.kernel,
        out_type=tuple(out_types) if gidx is not None else out_types[0],
        mesh=mesh,
        scratch_types=scratch,
    )
    def k(table_h, src_h, dst_h, zero_h, *rest):
        if gidx is not None:
            (gidx_h, out_h, gout_h, sidx, didx, rows, acc, gsem, ssem,
             gvidx, grbuf, g2sem) = rest
        else:
            out_h, sidx, didx, rows, acc, gsem, ssem = rest
        cid = lax.axis_index("c")
        sid = lax.axis_index("s")
        wid = cid * _NSUB + sid
        nck = jnp.where(cid == 0, nck0, nck1)
        base = jnp.where(cid == 0, sid * nck0, _NSUB * nck0 + sid * nck1)
        pltpu.sync_copy(zero_h.at[pl.ds(sid * rows_per_sub, rows_per_sub)],
                        acc.at[pl.ds(sid * rows_per_sub, rows_per_sub)])
        pltpu.sync_copy(src_h.at[pl.ds(base, nck_max)], sidx)
        pltpu.sync_copy(dst_h.at[pl.ds(base, nck_max)], didx)
        if gidx is not None:
            pltpu.sync_copy(gidx_h.at[pl.ds(wid, 1)], gvidx)
            pltpu.async_copy(table_h.at[gvidx.at[0]], grbuf, g2sem).wait()
            pltpu.sync_copy(grbuf, gout_h.at[pl.ds(wid * grows, grows)])
        plsc.subcore_barrier()

        def gather(c, b):
            pltpu.async_copy(table_h.at[sidx.at[c]], rows.at[b], gsem.at[b])

        def gwait(b):
            pltpu.make_async_copy(table_h.at[sidx.at[0]], rows.at[b],
                                  gsem.at[b]).wait()

        def swait(b):
            pltpu.make_async_copy(rows.at[b], acc.at[didx.at[0]],
                                  ssem.at[b]).wait()

        def scatter(c, b):
            pltpu.async_copy(rows.at[b], acc.at[didx.at[c]], ssem.at[b],
                             add=True)

        if use_ring:
            gather(0, 0)
            gather(1, 1)
            gwait(0)
            scatter(0, 0)

            @pl.loop(1, nck - 1, step=2)
            def _mid(c0):
                for j in range(2):
                    c = c0 + j
                    b = (1 + j) % 2
                    swait(1 - b)
                    gather(c + 1, 1 - b)
                    gwait(b)
                    scatter(c, b)

            gwait(1)
            scatter(nck - 1, 1)
            for b in range(nbuf):
                swait(b)
        else:
            @pl.loop(0, nck)
            def _chunk(c):
                pltpu.async_copy(table_h.at[sidx.at[c]], rows.at[0],
                                 gsem.at[0]).wait()
                pltpu.sync_copy(rows.at[0], acc.at[didx.at[c]], add=True)

        plsc.subcore_barrier()
        pltpu.sync_copy(acc.at[pl.ds(sid * rows_per_sub, rows_per_sub)],
                        out_h.at[cid, pl.ds(sid * rows_per_sub, rows_per_sub)])

    if gidx is not None:
        return k(table, src2d, dst2d, zeros, gidx2d)
    return k(table, src2d, dst2d, zeros)


def _pad_edges(src, dst, e_pad, dump_row, n_dump):
    e = src.shape[0]
    pad = e_pad - e
    src_p = jnp.concatenate([src, jnp.zeros((pad,), jnp.int32)])
    dump = dump_row + (jnp.arange(pad, dtype=jnp.int32) % n_dump)
    dst_p = jnp.concatenate([dst, dump])
    return src_p, dst_p


def _conv_body(x_ref, m_ref, wr_ref, wn_ref, b_ref, o_ref):
    acc = jnp.dot(x_ref[...], wr_ref[...], preferred_element_type=jnp.float32)
    acc = acc + jnp.dot(m_ref[...], wn_ref[...], preferred_element_type=jnp.float32)
    acc = acc + b_ref[...]
    o_ref[...] = jnp.maximum(acc, 0.0)


def _conv2_body(x_ref, ma_ref, mb_ref, wr_ref, wn_ref, b_ref, o_ref):
    acc = jnp.dot(x_ref[...], wr_ref[...], preferred_element_type=jnp.float32)
    m = ma_ref[...] + mb_ref[...]
    acc = acc + jnp.dot(m, wn_ref[...], preferred_element_type=jnp.float32)
    acc = acc + b_ref[...]
    o_ref[...] = jnp.maximum(acc, 0.0)


def _conv_relu(x, msgs, wr, wn, b, block_rows=None):
    n = x.shape[0]
    if block_rows is None:
        block_rows = 512 if n % 512 == 0 else 2000
    assert n % block_rows == 0, n
    grid = n // block_rows
    row_spec = pl.BlockSpec((block_rows, _HID), lambda i: (i, 0))
    w_spec = pl.BlockSpec((_HID, _HID), lambda i: (0, 0))
    b_spec = pl.BlockSpec((1, _HID), lambda i: (0, 0))
    body = _conv_body if len(msgs) == 1 else _conv2_body
    return pl.pallas_call(
        body,
        grid=(grid,),
        in_specs=[row_spec] * (1 + len(msgs)) + [w_spec, w_spec, b_spec],
        out_specs=row_spec,
        out_shape=jax.ShapeDtypeStruct((n, _HID), jnp.float32),
    )(x, *msgs, wr, wn, b.reshape(1, _HID))


def _final_body(x_ref, ma_ref, mb_ref, wr_ref, wn_ref, b_ref, lw_ref, lb_ref,
                o_ref):
    acc = jnp.dot(x_ref[...], wr_ref[...], preferred_element_type=jnp.float32)
    m = ma_ref[...] + mb_ref[...]
    acc = acc + jnp.dot(m, wn_ref[...], preferred_element_type=jnp.float32)
    acc = acc + b_ref[...]
    h = jnp.maximum(acc, 0.0)
    o_ref[...] = jnp.dot(h, lw_ref[...], preferred_element_type=jnp.float32) + lb_ref[...]


def _final_fused(x, msgs, wr, wn, b, lin_w, lin_b, block_rows=None):
    n = x.shape[0]
    if block_rows is None:
        block_rows = 512 if n % 512 == 0 else 2000
    assert n % block_rows == 0, n
    grid = n // block_rows
    row_spec = pl.BlockSpec((block_rows, _HID), lambda i: (i, 0))
    w_spec = pl.BlockSpec((_HID, _HID), lambda i: (0, 0))
    b_spec = pl.BlockSpec((1, _HID), lambda i: (0, 0))
    return pl.pallas_call(
        _final_body,
        grid=(grid,),
        in_specs=[row_spec, row_spec, row_spec, w_spec, w_spec, b_spec,
                  w_spec, b_spec],
        out_specs=row_spec,
        out_shape=jax.ShapeDtypeStruct((n, _HID), jnp.float32),
    )(x, *msgs, wr, wn, b.reshape(1, _HID), lin_w, lin_b.reshape(1, _HID))


def _knn_body(pos_ref, post_ref, o_ref, *, block_rows, npts_pad):
    i = pl.program_id(0)
    pr = pos_ref[...]
    pt = post_ref[...]
    d = jnp.zeros((block_rows, npts_pad), jnp.float32)
    for c in range(3):
        diff = pr[:, c:c + 1] - pt[c:c + 1, :]
        d = d + diff * diff
    row_id = lax.broadcasted_iota(jnp.int32, (block_rows, npts_pad), 0) + i * block_rows
    col_id = lax.broadcasted_iota(jnp.int32, (block_rows, npts_pad), 1)
    d = jnp.where(row_id == col_id, d + 1e10, d)
    cols = []
    for _ in range(_K):
        a = jnp.argmin(d, axis=1).astype(jnp.int32)
        cols.append(a[:, None])
        d = jnp.where(col_id == a[:, None], jnp.inf, d)
    cols.append(jnp.zeros((block_rows, 2), jnp.int32))
    o_ref[...] = jnp.concatenate(cols, axis=1)


def _knn_top6(pos_pad, post_pad, block_rows=256):
    npts_pad = post_pad.shape[1]
    grid = npts_pad // block_rows
    return pl.pallas_call(
        functools.partial(_knn_body, block_rows=block_rows, npts_pad=npts_pad),
        grid=(grid,),
        in_specs=[
            pl.BlockSpec((block_rows, 8), lambda i: (i, 0)),
            pl.BlockSpec((8, npts_pad), lambda i: (0, 0)),
        ],
        out_specs=pl.BlockSpec((block_rows, 8), lambda i: (i, 0)),
        out_shape=jax.ShapeDtypeStruct((npts_pad, 8), jnp.int32),
    )(pos_pad, post_pad)


def kernel(x, pos, edge_index, batch, d0_wr, d0_wn, d0_b, d1_wr, d1_wn, d1_b,
           u0_wr, u0_wn, u0_b, u1_wr, u1_wn, u1_b, lin_w, lin_b):
    n = x.shape[0]
    src0, dst0 = edge_index[0], edge_index[1]

    def _constants():
        p0 = jax.random.permutation(jax.random.key(100), n)[:_NP1]
        p1 = jax.random.permutation(jax.random.key(101), _NP1)[:_NP2]
        i1 = (_NP2 + (jnp.arange(_NP1, dtype=jnp.int32) % 96)).at[p1].set(
            jnp.arange(_NP2, dtype=jnp.int32))
        i0 = jnp.full((n,), _NP1, jnp.int32).at[p0].set(
            jnp.arange(_NP1, dtype=jnp.int32))
        dk = jnp.tile(jnp.arange(_NP1, dtype=jnp.int32), _K)
        perm0_pad = jnp.concatenate(
            [p0.astype(jnp.int32), jnp.zeros((_NP1_PAD - _NP1,), jnp.int32)])
        inv1_pad = jnp.concatenate(
            [i1, _NP2 + (jnp.arange(_NP1_PAD - _NP1, dtype=jnp.int32) % 96)])
        return (p0.astype(jnp.int32), p1.astype(jnp.int32), i0, i1, dk,
                perm0_pad, inv1_pad)
    try:
        with jax.ensure_compile_time_eval(), \
                jax.default_device(jax.devices("cpu")[0]):
            perm0, perm1, inv0, inv1, dstk, perm0_pad, inv1_pad = _constants()
    except Exception:
        perm0, perm1, inv0, inv1, dstk, perm0_pad, inv1_pad = _constants()

    n_acc = 10112
    e_pad0 = 163840
    src0_p, dst0_p = _pad_edges(src0, dst0, e_pad0, n, n_acc - n)
    parts0 = _sc_segsum_call(x, src0_p, dst0_p, n_acc, 64, 120, 40)
    h0 = _conv_relu(x, [parts0[0, :n], parts0[1, :n]], d0_wr, d0_wn, d0_b)

    pos1 = pos[perm0]
    pos1_pad = jnp.full((_NP1_PAD, 8), 1e6, jnp.float32)
    pos1_pad = pos1_pad.at[:_NP1, :3].set(pos1)
    post_pad = pos1_pad.T.reshape(8, _NP1_PAD) + 0.0
    idx_pad = _knn_top6(pos1_pad, post_pad)
    idx1 = idx_pad[:_NP1, :_K]

    e_pad1 = 16384
    idx1t = idx1.T.reshape(-1)
    gidx1 = perm0[idx1t]
    srck, dstk_p = _pad_edges(gidx1, dstk, e_pad1, _NP1, _NP1_PAD - _NP1)
    parts1, x1p = _sc_segsum_call(h0, srck, dstk_p, _NP1_PAD, 64,
                                  gidx=perm0_pad)
    h1 = _conv_relu(x1p, [parts1[0], parts1[1]], d1_wr, d1_wn, d1_b)[:_NP1]

    x2 = h1[perm1]
    t_tab = jnp.concatenate([x2, jnp.zeros((96, _HID), jnp.float32)], axis=0)
    srcu = inv1[idx1t]
    srcu_p, _ = _pad_edges(srcu, dstk, e_pad1, _NP1, _NP1_PAD - _NP1)
    parts2, xr0p = _sc_segsum_call(t_tab, srcu_p, dstk_p, _NP1_PAD, 64,
                                   gidx=inv1_pad)
    h2 = _conv_relu(xr0p, [parts2[0], parts2[1]], u0_wr, u0_wn, u0_b)[:_NP1]

    u_tab = jnp.concatenate([h2, jnp.zeros((1, _HID), jnp.float32)], axis=0)
    xg = u_tab[inv0]
    parts3 = _sc_segsum_call(xg, src0_p, dst0_p, n_acc, 64, 120, 40)
    return _final_fused(xg, [parts3[0, :n], parts3[1, :n]],
                        u1_wr, u1_wn, u1_b, lin_w, lin_b)


# speedup vs baseline: 1.7682x; 1.0002x over previous
import functools

import jax
import jax.numpy as jnp
from jax import lax
from jax.experimental import pallas as pl
from jax.experimental.pallas import tpu as pltpu
from jax.experimental.pallas import tpu_sc as plsc

_N = 10000
_HID = 128
_K = 6
_NP1 = 2500
_NP2 = 625
_NP1_PAD = 2560


_NSC = 2
_NSUB = 16
_NW = _NSC * _NSUB


def _sc_segsum_call(table, src_idx, dst_idx, n_out_pad, chunk,
                    nck0=None, nck1=None, gidx=None):
    e_pad = src_idx.shape[0]
    assert e_pad % chunk == 0
    ncks = e_pad // chunk
    if nck0 is None:
        nck0 = nck1 = ncks // _NW
    assert _NSUB * (nck0 + nck1) == ncks
    nck_max = max(nck0, nck1)
    rows_per_sub = n_out_pad // _NSUB
    src2d = src_idx.reshape(ncks, chunk)
    dst2d = dst_idx.reshape(ncks, chunk)
    extra = _NSUB * nck0 + (_NSUB - 1) * nck1 + nck_max - ncks
    if extra > 0:
        src2d = jnp.pad(src2d, ((0, extra), (0, 0)))
        dst2d = jnp.pad(dst2d, ((0, extra), (0, 0)))
    zeros = jnp.zeros((n_out_pad, _HID), jnp.float32)
    mesh = plsc.VectorSubcoreMesh(core_axis_name="c", subcore_axis_name="s")

    nbuf = 2
    use_ring = (nck0 % 2 == 0 and nck1 % 2 == 0 and min(nck0, nck1) >= 4)

    out_types = [jax.ShapeDtypeStruct((_NSC, n_out_pad, _HID), jnp.float32)]
    scratch = [
        pltpu.VMEM((nck_max, chunk), jnp.int32),
        pltpu.VMEM((nck_max, chunk), jnp.int32),
        pltpu.VMEM((nbuf, chunk, _HID), jnp.float32),
        pltpu.VMEM_SHARED((n_out_pad, _HID), jnp.float32),
        pltpu.SemaphoreType.DMA((nbuf,)),
        pltpu.SemaphoreType.DMA((nbuf,)),
    ]
    grows = n_out_pad // _NW
    if gidx is not None:
        out_types.append(jax.ShapeDtypeStruct((n_out_pad, _HID), jnp.float32))
        scratch += [
            pltpu.VMEM((1, grows), jnp.int32),
            pltpu.VMEM((grows, _HID), jnp.float32),
            pltpu.SemaphoreType.DMA,
        ]
        gidx2d = gidx.reshape(_NW, grows)

    @functools.partial(
        pl.kernel,
        out_type=tuple(out_types) if gidx is not None else out_types[0],
        mesh=mesh,
        scratch_types=scratch,
    )
    def k(table_h, src_h, dst_h, zero_h, *rest):
        if gidx is not None:
            (gidx_h, out_h, gout_h, sidx, didx, rows, acc, gsem, ssem,
             gvidx, grbuf, g2sem) = rest
        else:
            out_h, sidx, didx, rows, acc, gsem, ssem = rest
        cid = lax.axis_index("c")
        sid = lax.axis_index("s")
        wid = cid * _NSUB + sid
        nck = jnp.where(cid == 0, nck0, nck1)
        base = jnp.where(cid == 0, sid * nck0, _NSUB * nck0 + sid * nck1)
        pltpu.sync_copy(zero_h.at[pl.ds(sid * rows_per_sub, rows_per_sub)],
                        acc.at[pl.ds(sid * rows_per_sub, rows_per_sub)])
        pltpu.sync_copy(src_h.at[pl.ds(base, nck_max)], sidx)
        pltpu.sync_copy(dst_h.at[pl.ds(base, nck_max)], didx)
        if gidx is not None:
            pltpu.sync_copy(gidx_h.at[pl.ds(wid, 1)], gvidx)
            pltpu.async_copy(table_h.at[gvidx.at[0]], grbuf, g2sem).wait()
            pltpu.sync_copy(grbuf, gout_h.at[pl.ds(wid * grows, grows)])
        plsc.subcore_barrier()

        def gather(c, b):
            pltpu.async_copy(table_h.at[sidx.at[c]], rows.at[b], gsem.at[b])

        def gwait(b):
            pltpu.make_async_copy(table_h.at[sidx.at[0]], rows.at[b],
                                  gsem.at[b]).wait()

        def swait(b):
            pltpu.make_async_copy(rows.at[b], acc.at[didx.at[0]],
                                  ssem.at[b]).wait()

        def scatter(c, b):
            pltpu.async_copy(rows.at[b], acc.at[didx.at[c]], ssem.at[b],
                             add=True)

        if use_ring:
            gather(0, 0)
            gather(1, 1)
            gwait(0)
            scatter(0, 0)

            @pl.loop(1, nck - 1, step=2)
            def _mid(c0):
                for j in range(2):
                    c = c0 + j
                    b = (1 + j) % 2
                    swait(1 - b)
                    gather(c + 1, 1 - b)
                    gwait(b)
                    scatter(c, b)

            gwait(1)
            scatter(nck - 1, 1)
            for b in range(nbuf):
                swait(b)
        else:
            @pl.loop(0, nck)
            def _chunk(c):
                pltpu.async_copy(table_h.at[sidx.at[c]], rows.at[0],
                                 gsem.at[0]).wait()
                pltpu.sync_copy(rows.at[0], acc.at[didx.at[c]], add=True)

        plsc.subcore_barrier()
        pltpu.sync_copy(acc.at[pl.ds(sid * rows_per_sub, rows_per_sub)],
                        out_h.at[cid, pl.ds(sid * rows_per_sub, rows_per_sub)])

    if gidx is not None:
        return k(table, src2d, dst2d, zeros, gidx2d)
    return k(table, src2d, dst2d, zeros)


def _pad_edges(src, dst, e_pad, dump_row, n_dump):
    e = src.shape[0]
    pad = e_pad - e
    src_p = jnp.concatenate([src, jnp.zeros((pad,), jnp.int32)])
    dump = dump_row + (jnp.arange(pad, dtype=jnp.int32) % n_dump)
    dst_p = jnp.concatenate([dst, dump])
    return src_p, dst_p


def _conv_body(x_ref, m_ref, wr_ref, wn_ref, b_ref, o_ref):
    acc = jnp.dot(x_ref[...], wr_ref[...], preferred_element_type=jnp.float32)
    acc = acc + jnp.dot(m_ref[...], wn_ref[...], preferred_element_type=jnp.float32)
    acc = acc + b_ref[...]
    o_ref[...] = jnp.maximum(acc, 0.0)


def _conv2_body(x_ref, ma_ref, mb_ref, wr_ref, wn_ref, b_ref, o_ref):
    acc = jnp.dot(x_ref[...], wr_ref[...], preferred_element_type=jnp.float32)
    m = ma_ref[...] + mb_ref[...]
    acc = acc + jnp.dot(m, wn_ref[...], preferred_element_type=jnp.float32)
    acc = acc + b_ref[...]
    o_ref[...] = jnp.maximum(acc, 0.0)


def _conv_relu(x, msgs, wr, wn, b, block_rows=None):
    n = x.shape[0]
    if block_rows is None:
        block_rows = 512 if n % 512 == 0 else 2000
    assert n % block_rows == 0, n
    grid = n // block_rows
    row_spec = pl.BlockSpec((block_rows, _HID), lambda i: (i, 0))
    w_spec = pl.BlockSpec((_HID, _HID), lambda i: (0, 0))
    b_spec = pl.BlockSpec((1, _HID), lambda i: (0, 0))
    body = _conv_body if len(msgs) == 1 else _conv2_body
    return pl.pallas_call(
        body,
        grid=(grid,),
        in_specs=[row_spec] * (1 + len(msgs)) + [w_spec, w_spec, b_spec],
        out_specs=row_spec,
        out_shape=jax.ShapeDtypeStruct((n, _HID), jnp.float32),
    )(x, *msgs, wr, wn, b.reshape(1, _HID))


def _final_body(x_ref, ma_ref, mb_ref, wr_ref, wn_ref, b_ref, lw_ref, lb_ref,
                o_ref):
    acc = jnp.dot(x_ref[...], wr_ref[...], preferred_element_type=jnp.float32)
    m = ma_ref[...] + mb_ref[...]
    acc = acc + jnp.dot(m, wn_ref[...], preferred_element_type=jnp.float32)
    acc = acc + b_ref[...]
    h = jnp.maximum(acc, 0.0)
    o_ref[...] = jnp.dot(h, lw_ref[...], preferred_element_type=jnp.float32) + lb_ref[...]


def _final_fused(x, msgs, wr, wn, b, lin_w, lin_b, block_rows=None):
    n = x.shape[0]
    if block_rows is None:
        block_rows = 512 if n % 512 == 0 else 2000
    assert n % block_rows == 0, n
    grid = n // block_rows
    row_spec = pl.BlockSpec((block_rows, _HID), lambda i: (i, 0))
    w_spec = pl.BlockSpec((_HID, _HID), lambda i: (0, 0))
    b_spec = pl.BlockSpec((1, _HID), lambda i: (0, 0))
    return pl.pallas_call(
        _final_body,
        grid=(grid,),
        in_specs=[row_spec, row_spec, row_spec, w_spec, w_spec, b_spec,
                  w_spec, b_spec],
        out_specs=row_spec,
        out_shape=jax.ShapeDtypeStruct((n, _HID), jnp.float32),
    )(x, *msgs, wr, wn, b.reshape(1, _HID), lin_w, lin_b.reshape(1, _HID))


def _knn_body(pos_ref, post_ref, o_ref, *, block_rows, npts_pad):
    i = pl.program_id(0)
    pr = pos_ref[...]
    pt = post_ref[...]
    d = jnp.zeros((block_rows, npts_pad), jnp.float32)
    for c in range(3):
        diff = pr[:, c:c + 1] - pt[c:c + 1, :]
        d = d + diff * diff
    row_id = lax.broadcasted_iota(jnp.int32, (block_rows, npts_pad), 0) + i * block_rows
    col_id = lax.broadcasted_iota(jnp.int32, (block_rows, npts_pad), 1)
    d = jnp.where(row_id == col_id, d + 1e10, d)
    cols = []
    for _ in range(_K):
        a = jnp.argmin(d, axis=1).astype(jnp.int32)
        cols.append(a[:, None])
        d = jnp.where(col_id == a[:, None], jnp.inf, d)
    cols.append(jnp.zeros((block_rows, 2), jnp.int32))
    o_ref[...] = jnp.concatenate(cols, axis=1)


def _knn_top6(pos_pad, post_pad, block_rows=256):
    npts_pad = post_pad.shape[1]
    grid = npts_pad // block_rows
    return pl.pallas_call(
        functools.partial(_knn_body, block_rows=block_rows, npts_pad=npts_pad),
        grid=(grid,),
        in_specs=[
            pl.BlockSpec((block_rows, 8), lambda i: (i, 0)),
            pl.BlockSpec((8, npts_pad), lambda i: (0, 0)),
        ],
        out_specs=pl.BlockSpec((block_rows, 8), lambda i: (i, 0)),
        out_shape=jax.ShapeDtypeStruct((npts_pad, 8), jnp.int32),
    )(pos_pad, post_pad)


def kernel(x, pos, edge_index, batch, d0_wr, d0_wn, d0_b, d1_wr, d1_wn, d1_b,
           u0_wr, u0_wn, u0_b, u1_wr, u1_wn, u1_b, lin_w, lin_b):
    n = x.shape[0]
    src0, dst0 = edge_index[0], edge_index[1]

    def _constants():
        p0 = jax.random.permutation(jax.random.key(100), n)[:_NP1]
        p1 = jax.random.permutation(jax.random.key(101), _NP1)[:_NP2]
        i1 = (_NP2 + (jnp.arange(_NP1, dtype=jnp.int32) % 96)).at[p1].set(
            jnp.arange(_NP2, dtype=jnp.int32))
        i0 = jnp.full((n,), _NP1, jnp.int32).at[p0].set(
            jnp.arange(_NP1, dtype=jnp.int32))
        dk = jnp.tile(jnp.arange(_NP1, dtype=jnp.int32), _K)
        perm0_pad = jnp.concatenate(
            [p0.astype(jnp.int32), jnp.zeros((_NP1_PAD - _NP1,), jnp.int32)])
        inv1_pad = jnp.concatenate(
            [i1, _NP2 + (jnp.arange(_NP1_PAD - _NP1, dtype=jnp.int32) % 96)])
        return (p0.astype(jnp.int32), p1.astype(jnp.int32), i0, i1, dk,
                perm0_pad, inv1_pad)
    try:
        with jax.ensure_compile_time_eval(), \
                jax.default_device(jax.devices("cpu")[0]):
            perm0, perm1, inv0, inv1, dstk, perm0_pad, inv1_pad = _constants()
    except Exception:
        perm0, perm1, inv0, inv1, dstk, perm0_pad, inv1_pad = _constants()

    n_acc = 10112
    e_pad0 = 163840
    src0_p, dst0_p = _pad_edges(src0, dst0, e_pad0, n, n_acc - n)
    parts0 = _sc_segsum_call(x, src0_p, dst0_p, n_acc, 64, 120, 40)
    h0 = _conv_relu(x, [parts0[0, :n], parts0[1, :n]], d0_wr, d0_wn, d0_b)

    pos1 = pos[perm0]
    pos1_pad = jnp.full((_NP1_PAD, 8), 1e6, jnp.float32)
    pos1_pad = pos1_pad.at[:_NP1, :3].set(pos1)
    post_pad = pos1_pad.T.reshape(8, _NP1_PAD) + 0.0
    idx_pad = _knn_top6(pos1_pad, post_pad)
    idx1 = idx_pad[:_NP1, :_K]

    e_pad1 = 16384
    idx1t = idx1.T.reshape(-1)
    gidx1 = perm0[idx1t]
    srck, dstk_p = _pad_edges(gidx1, dstk, e_pad1, _NP1, _NP1_PAD - _NP1)
    parts1, x1p = _sc_segsum_call(h0, srck, dstk_p, _NP1_PAD, 32, 24, 8,
                                  gidx=perm0_pad)
    h1 = _conv_relu(x1p, [parts1[0], parts1[1]], d1_wr, d1_wn, d1_b)[:_NP1]

    x2 = h1[perm1]
    t_tab = jnp.concatenate([x2, jnp.zeros((96, _HID), jnp.float32)], axis=0)
    srcu = inv1[idx1t]
    srcu_p, _ = _pad_edges(srcu, dstk, e_pad1, _NP1, _NP1_PAD - _NP1)
    parts2, xr0p = _sc_segsum_call(t_tab, srcu_p, dstk_p, _NP1_PAD, 32, 24, 8,
                                   gidx=inv1_pad)
    h2 = _conv_relu(xr0p, [parts2[0], parts2[1]], u0_wr, u0_wn, u0_b)[:_NP1]

    u_tab = jnp.concatenate([h2, jnp.zeros((1, _HID), jnp.float32)], axis=0)
    xg = u_tab[inv0]
    parts3 = _sc_segsum_call(xg, src0_p, dst0_p, n_acc, 64, 120, 40)
    return _final_fused(xg, [parts3[0, :n], parts3[1, :n]],
                        u1_wr, u1_wn, u1_b, lin_w, lin_b)
